# baseline (speedup 1.0000x reference)
import functools

import jax
import jax.numpy as jnp
from jax import lax
from jax.experimental import pallas as pl
from jax.experimental.pallas import tpu as pltpu
from jax.experimental.pallas import tpu_sc as plsc

_NUM_EMBEDDINGS = 100000
_SEQ_LEN = 200
_EMB_DIM = 64
_BATCH = 4096

_NW = 32
_BW = _BATCH // _NW


def _position_embedding_host():
    even_index = jnp.arange(0, _EMB_DIM, 2, dtype=jnp.float32)
    denominator = jnp.power(10000.0, even_index / _EMB_DIM)
    positions = jnp.arange(0, _SEQ_LEN, dtype=jnp.float32).reshape(_SEQ_LEN, 1)
    even_pe = jnp.sin(positions / denominator)
    odd_pe = jnp.cos(positions / denominator)
    stacked = jnp.stack([even_pe, odd_pe], axis=2)
    return stacked.reshape(_SEQ_LEN, _EMB_DIM)


def _sc_body(table_hbm, idx_hbm, pos_hbm, out_hbm,
             idx_v, pos_v, buf_a, buf_b, tbuf, sem_a, sem_b):
    nc = 2
    wid = lax.axis_index("s") * nc + lax.axis_index("c")
    last_even = _SEQ_LEN - 2
    ii = lax.iota(jnp.int32, 16)
    bvecs = [16 * c + ii for c in range(_BW // 16)]

    pltpu.sync_copy(idx_hbm.at[:, pl.ds(wid * _BW, _BW)], idx_v)
    pltpu.sync_copy(pos_hbm, pos_v)

    def fire(s, buf, sem):
        vals = tuple(
            pos_v[s, pl.ds(16 * c, 16)] for c in range(_EMB_DIM // 16)
        )

        def rep(r, vs):
            for c in range(_EMB_DIM // 16):
                buf[r, pl.ds(16 * c, 16)] = vs[c]
            return vs

        lax.fori_loop(0, _BW, rep, vals, unroll=4)
        return pltpu.async_copy(table_hbm.at[idx_v.at[s]], buf, sem, add=True)

    def flush(s, buf):
        def erow(e, c2):
            evec = jnp.full((16,), e, jnp.int32)
            for c in range(_BW // 16):
                tbuf[e // 8, e % 8, pl.ds(16 * c, 16)] = plsc.load_gather(
                    buf, [bvecs[c], evec])
            return c2

        lax.fori_loop(0, _EMB_DIM, erow, 0, unroll=2)
        pltpu.sync_copy(tbuf, out_hbm.at[s, :, wid])

    fire(0, buf_a, sem_a)

    def body(so, carry):
        s = 2 * so
        fire(s + 1, buf_b, sem_b)
        pltpu.make_async_copy(table_hbm.at[idx_v.at[0]], buf_a, sem_a).wait()
        flush(s, buf_a)
        fire(jnp.minimum(s + 2, last_even), buf_a, sem_a)
        pltpu.make_async_copy(table_hbm.at[idx_v.at[0]], buf_b, sem_b).wait()
        flush(s + 1, buf_b)
        return carry

    lax.fori_loop(0, _SEQ_LEN // 2, body, 0)
    pltpu.make_async_copy(table_hbm.at[idx_v.at[0]], buf_a, sem_a).wait()


@jax.jit
def kernel(input, table):
    pos = _position_embedding_host()
    table = jnp.pad(table, ((0, 0), (0, 128 - _EMB_DIM))).reshape(
        2 * _NUM_EMBEDDINGS, _EMB_DIM
    )
    idx_t = (input * 2).T

    mesh = plsc.VectorSubcoreMesh(core_axis_name="c", subcore_axis_name="s")
    out5d = pl.kernel(
        _sc_body,
        out_type=jax.ShapeDtypeStruct(
            (_SEQ_LEN, _EMB_DIM // 8, _BATCH // 128, 8, 128), jnp.float32
        ),
        mesh=mesh,
        scratch_types=[
            pltpu.VMEM((_SEQ_LEN, _BW), jnp.int32),
            pltpu.VMEM((_SEQ_LEN, _EMB_DIM), jnp.float32),
            pltpu.VMEM((_BW, _EMB_DIM), jnp.float32),
            pltpu.VMEM((_BW, _EMB_DIM), jnp.float32),
            pltpu.VMEM((_EMB_DIM // 8, 8, 128), jnp.float32),
            pltpu.SemaphoreType.DMA,
            pltpu.SemaphoreType.DMA,
        ],
        compiler_params=pltpu.CompilerParams(
            use_tc_tiling_on_sc=False, needs_layout_passes=False
        ),
    )(table, idx_t, pos)
    return out5d.transpose(2, 4, 0, 1, 3).reshape(_BATCH, _SEQ_LEN, _EMB_DIM)
